# Initial kernel scaffold; baseline (speedup 1.0000x reference)
import jax
import jax.numpy as jnp
from jax.experimental import pallas as pl


def kernel(mol_1_graph, mol_1_nodes, mol_2_graph, mol_2_nodes, params):
    raise NotImplementedError("write your pallas kernel here")


# speedup vs baseline: 5.8736x; 5.8736x over previous
import functools

import jax
import jax.numpy as jnp
from jax import lax
from jax.experimental import pallas as pl
from jax.experimental.pallas import tpu as pltpu
from jax.experimental.pallas import tpu_sc as plsc

N = 10000
E = 160000
D_IN = 256
HID = 20
HP = 32
D_OUT = 128
NN = 2 * N

NC, NS = 2, 16
NW = NC * NS
CH = 128
E2 = 2 * E
K = -(-E2 // (NW * CH))
EP = NW * K * CH
EPAD = EP - E2

ACC_ROWS = 20480
ZROWS = 160
ACC_PER_SUB = ACC_ROWS // NS


def _seg_body(y_hbm, src_hbm, dst_hbm, p_hbm, acc_s, src_v, dst_v, rows_v,
              zbuf_v, sem):
    c = lax.axis_index("c")
    s = lax.axis_index("s")
    wid = c * NS + s

    pltpu.sync_copy(src_hbm.at[wid], src_v)
    pltpu.sync_copy(dst_hbm.at[wid], dst_v)

    zeros16 = jnp.zeros((16,), jnp.float32)

    def _zfill(i, carry):
        zbuf_v[i // 2, pl.ds((i % 2) * 16, 16)] = zeros16
        return carry

    lax.fori_loop(0, ZROWS * 2, _zfill, 0)
    for t in range(ACC_PER_SUB // ZROWS):
        pltpu.sync_copy(zbuf_v,
                        acc_s.at[pl.ds(s * ACC_PER_SUB + t * ZROWS, ZROWS)])
    plsc.subcore_barrier()

    def _chunk(j, carry):
        pltpu.async_copy(y_hbm.at[src_v.at[j]], rows_v, sem).wait()
        pltpu.sync_copy(rows_v, acc_s.at[dst_v.at[j]], add=True)
        return carry

    lax.fori_loop(0, K, _chunk, 0)
    plsc.subcore_barrier()

    pltpu.sync_copy(acc_s.at[pl.ds(s * ACC_PER_SUB, ACC_PER_SUB)],
                    p_hbm.at[c, pl.ds(s * ACC_PER_SUB, ACC_PER_SUB)])


@functools.lru_cache(maxsize=1)
def _seg_kernel():
    return pl.kernel(
        _seg_body,
        out_type=jax.ShapeDtypeStruct((NC, ACC_ROWS, HP), jnp.float32),
        mesh=plsc.VectorSubcoreMesh(core_axis_name="c", subcore_axis_name="s",
                                    num_cores=NC, num_subcores=NS),
        scratch_types=[
            pltpu.VMEM_SHARED((ACC_ROWS, HP), jnp.float32),
            pltpu.VMEM((K, CH), jnp.int32),
            pltpu.VMEM((K, CH), jnp.int32),
            pltpu.VMEM((CH, HP), jnp.float32),
            pltpu.VMEM((ZROWS, HP), jnp.float32),
            pltpu.SemaphoreType.DMA,
        ],
        compiler_params=pltpu.CompilerParams(use_tc_tiling_on_sc=False),
    )


def _seg(y, src3, dst3):
    return _seg_kernel()(y, src3, dst3)


_BLK_A = 2000


def _stage_a_body(x_ref, wr_ref, wq_ref, t_ref, r_ref):
    x = x_ref[...]
    t_ref[...] = jnp.dot(x, wr_ref[...], preferred_element_type=jnp.float32)
    r_ref[...] = jnp.dot(x, wq_ref[...], preferred_element_type=jnp.float32)


def _stage_a(x, wr, wq):
    grid = (NN // _BLK_A,)
    return pl.pallas_call(
        _stage_a_body,
        grid=grid,
        in_specs=[
            pl.BlockSpec((_BLK_A, D_IN), lambda i: (i, 0)),
            pl.BlockSpec((D_IN, HP), lambda i: (0, 0)),
            pl.BlockSpec((D_IN, HP), lambda i: (0, 0)),
        ],
        out_specs=[pl.BlockSpec((_BLK_A, HP), lambda i: (i, 0))] * 2,
        out_shape=[jax.ShapeDtypeStruct((NN, HP), jnp.float32)] * 2,
    )(x, wr, wq)


_BLK_B = 2000


def _stage_b1_body(p_ref, a_ref, b_ref, wr_ref, h_ref, t_ref):
    h = jnp.maximum(p_ref[0] + p_ref[1] + b_ref[...] + a_ref[...], 0.0)
    h_ref[...] = h
    t_ref[...] = jnp.dot(h, wr_ref[...], preferred_element_type=jnp.float32)


def _stage_b2_body(p_ref, a_ref, b_ref, wq_ref, wr_ref, h_ref, t_ref):
    root = jnp.dot(a_ref[...], wq_ref[...], preferred_element_type=jnp.float32)
    h = jnp.maximum(p_ref[0] + p_ref[1] + b_ref[...] + root, 0.0)
    h_ref[...] = h
    t_ref[...] = jnp.dot(h, wr_ref[...], preferred_element_type=jnp.float32)


def _stage_b3_body(p_ref, a_ref, b_ref, wq_ref, h_ref):
    root = jnp.dot(a_ref[...], wq_ref[...], preferred_element_type=jnp.float32)
    h_ref[...] = jnp.maximum(p_ref[0] + p_ref[1] + b_ref[...] + root, 0.0)


def _stage_b1(p, a, b, wr):
    grid = (NN // _BLK_B,)
    return pl.pallas_call(
        _stage_b1_body,
        grid=grid,
        in_specs=[
            pl.BlockSpec((NC, _BLK_B, HP), lambda i: (0, i, 0)),
            pl.BlockSpec((_BLK_B, HP), lambda i: (i, 0)),
            pl.BlockSpec((1, HP), lambda i: (0, 0)),
            pl.BlockSpec((HP, HP), lambda i: (0, 0)),
        ],
        out_specs=[pl.BlockSpec((_BLK_B, HP), lambda i: (i, 0))] * 2,
        out_shape=[jax.ShapeDtypeStruct((NN, HP), jnp.float32)] * 2,
    )(p, a, b, wr)


def _stage_b2(p, a, b, wq, wr):
    grid = (NN // _BLK_B,)
    return pl.pallas_call(
        _stage_b2_body,
        grid=grid,
        in_specs=[
            pl.BlockSpec((NC, _BLK_B, HP), lambda i: (0, i, 0)),
            pl.BlockSpec((_BLK_B, HP), lambda i: (i, 0)),
            pl.BlockSpec((1, HP), lambda i: (0, 0)),
            pl.BlockSpec((HP, HP), lambda i: (0, 0)),
            pl.BlockSpec((HP, HP), lambda i: (0, 0)),
        ],
        out_specs=[pl.BlockSpec((_BLK_B, HP), lambda i: (i, 0))] * 2,
        out_shape=[jax.ShapeDtypeStruct((NN, HP), jnp.float32)] * 2,
    )(p, a, b, wq, wr)


def _stage_b3(p, a, b, wq):
    grid = (NN // _BLK_B,)
    return pl.pallas_call(
        _stage_b3_body,
        grid=grid,
        in_specs=[
            pl.BlockSpec((NC, _BLK_B, HP), lambda i: (0, i, 0)),
            pl.BlockSpec((_BLK_B, HP), lambda i: (i, 0)),
            pl.BlockSpec((1, HP), lambda i: (0, 0)),
            pl.BlockSpec((HP, HP), lambda i: (0, 0)),
        ],
        out_specs=pl.BlockSpec((_BLK_B, HP), lambda i: (i, 0)),
        out_shape=jax.ShapeDtypeStruct((NN, HP), jnp.float32),
    )(p, a, b, wq)


_BLK_C = 1000
_NBLK_C = NN // _BLK_C
_MOL_BLKS = N // _BLK_C


def _stage_c_body(p_ref, h3_ref, wr_ref, bo_ref, wq_ref, wl1_ref, bl1_ref,
                  wl2_ref, bl2_ref, wh1_ref, bh1_ref, wh2_ref, bh2_ref,
                  wh3_ref, bh3_ref, out_ref, acc):
    i = pl.program_id(0)
    agg = p_ref[0] + p_ref[1]
    h4 = jnp.maximum(
        jnp.dot(agg, wr_ref[...], preferred_element_type=jnp.float32)
        + bo_ref[...]
        + jnp.dot(h3_ref[...], wq_ref[...], preferred_element_type=jnp.float32),
        0.0,
    )
    bs = jnp.sum(h4, axis=0, keepdims=True)

    @pl.when(i == 0)
    def _():
        acc[0:1, :] = bs

    @pl.when((i > 0) & (i < _MOL_BLKS))
    def _():
        acc[0:1, :] = acc[0:1, :] + bs

    @pl.when(i == _MOL_BLKS)
    def _():
        acc[1:2, :] = bs

    @pl.when(i > _MOL_BLKS)
    def _():
        acc[1:2, :] = acc[1:2, :] + bs

    @pl.when(i == _NBLK_C - 1)
    def _():
        m = jnp.maximum(
            jnp.dot(acc[...], wl1_ref[...], preferred_element_type=jnp.float32)
            + bl1_ref[...], 0.0)
        m = jnp.maximum(
            jnp.dot(m, wl2_ref[...], preferred_element_type=jnp.float32)
            + bl2_ref[...], 0.0)
        z = (jnp.dot(m[0:1, :], wh1_ref[0:D_OUT, :],
                     preferred_element_type=jnp.float32)
             + jnp.dot(m[1:2, :], wh1_ref[D_OUT:2 * D_OUT, :],
                       preferred_element_type=jnp.float32)
             + bh1_ref[...])
        z = jnp.maximum(z, 0.0)
        z = jnp.maximum(
            jnp.dot(z, wh2_ref[...], preferred_element_type=jnp.float32)
            + bh2_ref[...], 0.0)
        z = (jnp.dot(z, wh3_ref[...], preferred_element_type=jnp.float32)
             + bh3_ref[...])
        out_ref[...] = 1.0 / (1.0 + jnp.exp(-z))


def _stage_c(p, h3, wr, bo, wq, wl1, bl1, wl2, bl2, wh1, bh1, wh2, bh2, wh3,
             bh3):
    grid = (_NBLK_C,)

    def _full(shape):
        nd = len(shape)
        return pl.BlockSpec(shape, lambda i, _nd=nd: (0,) * _nd)

    return pl.pallas_call(
        _stage_c_body,
        grid=grid,
        in_specs=[
            pl.BlockSpec((NC, _BLK_C, HP), lambda i: (0, i, 0)),
            pl.BlockSpec((_BLK_C, HP), lambda i: (i, 0)),
            _full((HP, D_OUT)),
            _full((1, D_OUT)),
            _full((HP, D_OUT)),
            _full((D_OUT, D_OUT)),
            _full((1, D_OUT)),
            _full((D_OUT, D_OUT)),
            _full((1, D_OUT)),
            _full((2 * D_OUT, 10)),
            _full((1, 10)),
            _full((10, 10)),
            _full((1, 10)),
            _full((10, 1)),
            _full((1, 1)),
        ],
        out_specs=pl.BlockSpec((1, 1), lambda i: (0, 0)),
        out_shape=jax.ShapeDtypeStruct((1, 1), jnp.float32),
        scratch_shapes=[pltpu.VMEM((2, D_OUT), jnp.float32)],
    )(p, h3, wr, bo, wq, wl1, bl1, wl2, bl2, wh1, bh1, wh2, bh2, wh3, bh3)


def _pad_cols(w, width=HP):
    return jnp.pad(w, ((0, 0), (0, width - w.shape[1])))


def _pad_rows(w, height=HP):
    return jnp.pad(w, ((0, height - w.shape[0]), (0, 0)))


def kernel(mol_1_graph, mol_1_nodes, mol_2_graph, mol_2_nodes, params):
    pr = params
    wr_in = _pad_cols(pr['conv_in']['W_rel'])
    wq_in = _pad_cols(pr['conv_in']['W_root'])
    b_in = _pad_cols(pr['conv_in']['b'][None])
    li1, li2 = pr['conv_internal']
    wr1 = _pad_cols(_pad_rows(li1['W_rel']))
    wq1 = _pad_cols(_pad_rows(li1['W_root']))
    b1 = _pad_cols(li1['b'][None])
    wr2 = _pad_cols(_pad_rows(li2['W_rel']))
    wq2 = _pad_cols(_pad_rows(li2['W_root']))
    b2 = _pad_cols(li2['b'][None])
    wr_out = _pad_rows(pr['conv_out']['W_rel'])
    wq_out = _pad_rows(pr['conv_out']['W_root'])
    b_out = pr['conv_out']['b'][None]
    lo1, lo2 = pr['linear_output']
    wh1 = pr['linear_1']['W']
    bh1 = pr['linear_1']['b'][None]
    wh2 = pr['linear_2']['W']
    bh2 = pr['linear_2']['b'][None]
    wh3 = pr['linear_3']['W']
    bh3 = pr['linear_3']['b'][None]

    x = jnp.concatenate([mol_1_nodes, mol_2_nodes], axis=0)
    src = jnp.concatenate([
        mol_1_graph[0], mol_2_graph[0] + N,
        jnp.zeros((EPAD,), jnp.int32),
    ])
    dst = jnp.concatenate([
        mol_1_graph[1], mol_2_graph[1] + N,
        jnp.full((EPAD,), NN, jnp.int32),
    ])
    src3 = src.reshape(NW, K, CH)
    dst3 = dst.reshape(NW, K, CH)

    t0, r0 = _stage_a(x, wr_in, wq_in)
    p0 = _seg(t0, src3, dst3)
    h1, t1 = _stage_b1(p0, r0, b_in, wr1)
    p1 = _seg(t1, src3, dst3)
    h2, t2 = _stage_b2(p1, h1, b1, wq1, wr2)
    p2 = _seg(t2, src3, dst3)
    h3 = _stage_b3(p2, h2, b2, wq2)
    p3 = _seg(h3, src3, dst3)
    out = _stage_c(p3, h3, wr_out, b_out, wq_out,
                   lo1['W'], lo1['b'][None], lo2['W'], lo2['b'][None],
                   wh1, bh1, wh2, bh2, wh3, bh3)
    return out.reshape((1,))
